# baseline (speedup 1.0000x reference)
import functools

import jax
import jax.numpy as jnp
from jax import lax
from jax.experimental import pallas as pl
from jax.experimental.pallas import tpu as pltpu
from jax.experimental.pallas import tpu_sc as plsc

NC, NS, L = 2, 16, 16
NW = NC * NS
UNROLL = 16


@jax.jit
def _sc_embed(xt, tt):
    F, B = xt.shape
    E, V = tt.shape
    e_per_w = E // NW
    n_steps = B // (L * UNROLL)
    mesh = plsc.VectorSubcoreMesh(core_axis_name="c", subcore_axis_name="s")

    @functools.partial(
        pl.kernel,
        out_type=jax.ShapeDtypeStruct((F, E, B), jnp.float32),
        mesh=mesh,
        scratch_types=[
            pltpu.VMEM((V,), jnp.float32),
            pltpu.VMEM((B,), jnp.int32),
            pltpu.VMEM((B,), jnp.int32),
            pltpu.VMEM((B,), jnp.float32),
            pltpu.VMEM((B,), jnp.float32),
            pltpu.SemaphoreType.DMA,
            pltpu.SemaphoreType.DMA,
            pltpu.SemaphoreType.DMA,
        ],
        compiler_params=pltpu.CompilerParams(needs_layout_passes=False),
    )
    def k(xt_hbm, tt_hbm, out_hbm, row_v, idx0, idx1, ob0, ob1,
          rsem, isem, osem):
        wid = lax.axis_index("s") * NC + lax.axis_index("c")
        idx_bufs = (idx0, idx1)
        out_bufs = (ob0, ob1)

        def row_copy(j):
            c = pltpu.make_async_copy(
                tt_hbm.at[wid * e_per_w + j], row_v, rsem)
            c.start()
            return c

        def idx_copy(f):
            c = pltpu.make_async_copy(xt_hbm.at[f], idx_bufs[f % 2], isem)
            c.start()
            return c

        def out_copy(f, e):
            c = pltpu.make_async_copy(
                out_bufs[f % 2], out_hbm.at[f, e], osem)
            c.start()
            return c

        rc = row_copy(0)
        ics = [idx_copy(0), idx_copy(1)]
        for j in range(e_per_w):
            e = wid * e_per_w + j
            rc.wait()
            if j > 0:
                ics = [idx_copy(0), idx_copy(1)]
            ocs = [None] * F
            for f in range(F):
                ics[f].wait()
                if f >= 2:
                    ocs[f - 2].wait()
                src = idx_bufs[f % 2]
                dst = out_bufs[f % 2]

                @plsc.parallel_loop(0, B, step=L, unroll=UNROLL)
                def gbody(i, src=src, dst=dst):
                    idx = src[pl.ds(i, L)]
                    dst[pl.ds(i, L)] = plsc.load_gather(row_v, [idx])
                if f + 2 < F:
                    ics.append(idx_copy(f + 2))
                if f == F - 1 and j + 1 < e_per_w:
                    rc = row_copy(j + 1)
                ocs[f] = out_copy(f, e)
            ocs[F - 2].wait()
            ocs[F - 1].wait()

    return k(xt, tt)


def kernel(x, table):
    out_t = _sc_embed(x.T, table.T)
    return out_t.transpose(2, 0, 1)


# speedup vs baseline: 1.0377x; 1.0377x over previous
import functools

import jax
import jax.numpy as jnp
from jax import lax
from jax.experimental import pallas as pl
from jax.experimental.pallas import tpu as pltpu
from jax.experimental.pallas import tpu_sc as plsc

NC, NS, L = 2, 16, 16
NW = NC * NS
UNROLL = 8


@jax.jit
def _sc_embed(xt, tt):
    F, B = xt.shape
    E, V = tt.shape
    e_per_w = E // NW
    n_steps = B // (L * UNROLL)
    mesh = plsc.VectorSubcoreMesh(core_axis_name="c", subcore_axis_name="s")

    @functools.partial(
        pl.kernel,
        out_type=jax.ShapeDtypeStruct((F, E, B), jnp.float32),
        mesh=mesh,
        scratch_types=[
            pltpu.VMEM((V,), jnp.float32),
            pltpu.VMEM((B,), jnp.int32),
            pltpu.VMEM((B,), jnp.int32),
            pltpu.VMEM((B,), jnp.float32),
            pltpu.VMEM((B,), jnp.float32),
            pltpu.SemaphoreType.DMA,
            pltpu.SemaphoreType.DMA,
            pltpu.SemaphoreType.DMA,
        ],
        compiler_params=pltpu.CompilerParams(needs_layout_passes=False),
    )
    def k(xt_hbm, tt_hbm, out_hbm, row_v, idx0, idx1, ob0, ob1,
          rsem, isem, osem):
        wid = lax.axis_index("s") * NC + lax.axis_index("c")
        idx_bufs = (idx0, idx1)
        out_bufs = (ob0, ob1)

        def row_copy(j):
            c = pltpu.make_async_copy(
                tt_hbm.at[wid * e_per_w + j], row_v, rsem)
            c.start()
            return c

        def idx_copy(f):
            c = pltpu.make_async_copy(xt_hbm.at[f], idx_bufs[f % 2], isem)
            c.start()
            return c

        def out_copy(f, e):
            c = pltpu.make_async_copy(
                out_bufs[f % 2], out_hbm.at[f, e], osem)
            c.start()
            return c

        rc = row_copy(0)
        ics = [idx_copy(0), idx_copy(1)]
        for j in range(e_per_w):
            e = wid * e_per_w + j
            rc.wait()
            if j > 0:
                ics = [idx_copy(0), idx_copy(1)]
            ocs = [None] * F
            for f in range(F):
                ics[f].wait()
                if f >= 2:
                    ocs[f - 2].wait()
                src = idx_bufs[f % 2]
                dst = out_bufs[f % 2]

                @plsc.parallel_loop(0, B, step=L, unroll=UNROLL)
                def gbody(i, src=src, dst=dst):
                    idx = src[pl.ds(i, L)]
                    dst[pl.ds(i, L)] = plsc.load_gather(row_v, [idx])
                if f + 2 < F:
                    ics.append(idx_copy(f + 2))
                if f == F - 1 and j + 1 < e_per_w:
                    rc = row_copy(j + 1)
                ocs[f] = out_copy(f, e)
            ocs[F - 2].wait()
            ocs[F - 1].wait()

    return k(xt, tt)


def kernel(x, table):
    out_t = _sc_embed(x.T, table.T)
    return out_t.transpose(2, 0, 1)


# speedup vs baseline: 1.1279x; 1.0869x over previous
import functools

import jax
import jax.numpy as jnp
from jax import lax
from jax.experimental import pallas as pl
from jax.experimental.pallas import tpu as pltpu
from jax.experimental.pallas import tpu_sc as plsc

NC, NS, L = 2, 16, 16
NW = NC * NS
UNROLL = 8


@jax.jit
def _sc_embed(xt, tt):
    F, B = xt.shape
    E, V = tt.shape
    e_per_w = E // NW
    n_steps = B // (L * UNROLL)
    mesh = plsc.VectorSubcoreMesh(core_axis_name="c", subcore_axis_name="s")

    @functools.partial(
        pl.kernel,
        out_type=jax.ShapeDtypeStruct((F, E, B), jnp.float32),
        mesh=mesh,
        scratch_types=[
            pltpu.VMEM((V,), jnp.float32),
            pltpu.VMEM((B,), jnp.int32),
            pltpu.VMEM((B,), jnp.int32),
            pltpu.VMEM((B,), jnp.int32),
            pltpu.VMEM((B,), jnp.float32),
            pltpu.VMEM((B,), jnp.float32),
            pltpu.VMEM((B,), jnp.float32),
            pltpu.SemaphoreType.DMA,
            pltpu.SemaphoreType.DMA,
            pltpu.SemaphoreType.DMA,
        ],
        compiler_params=pltpu.CompilerParams(needs_layout_passes=False),
    )
    def k(xt_hbm, tt_hbm, out_hbm, row_v, idx0, idx1, idx2, ob0, ob1, ob2,
          rsem, isem, osem):
        wid = lax.axis_index("s") * NC + lax.axis_index("c")
        idx_bufs = (idx0, idx1, idx2)
        out_bufs = (ob0, ob1, ob2)

        def row_copy(j):
            c = pltpu.make_async_copy(
                tt_hbm.at[wid * e_per_w + j], row_v, rsem)
            c.start()
            return c

        def idx_copy(f):
            c = pltpu.make_async_copy(xt_hbm.at[f], idx_bufs[f % 3], isem)
            c.start()
            return c

        def out_copy(f, e):
            c = pltpu.make_async_copy(
                out_bufs[f % 3], out_hbm.at[f, e], osem)
            c.start()
            return c

        rc = row_copy(0)
        ics = [idx_copy(0), idx_copy(1)]
        for j in range(e_per_w):
            e = wid * e_per_w + j
            rc.wait()
            if j > 0:
                ics = [idx_copy(0), idx_copy(1)]
            ocs = [None] * F
            for f in range(F):
                ics[f].wait()
                if f + 2 < F:
                    ics.append(idx_copy(f + 2))
                if f >= 3:
                    ocs[f - 3].wait()
                src = idx_bufs[f % 3]
                dst = out_bufs[f % 3]

                @plsc.parallel_loop(0, B, step=L, unroll=UNROLL)
                def gbody(i, src=src, dst=dst):
                    idx = src[pl.ds(i, L)]
                    dst[pl.ds(i, L)] = plsc.load_gather(row_v, [idx])
                if f == F - 1 and j + 1 < e_per_w:
                    rc = row_copy(j + 1)
                ocs[f] = out_copy(f, e)
            ocs[F - 3].wait()
            ocs[F - 2].wait()
            ocs[F - 1].wait()

    return k(xt, tt)


def kernel(x, table):
    out_t = _sc_embed(x.T, table.T)
    return out_t.transpose(2, 0, 1)


# speedup vs baseline: 1.1433x; 1.0137x over previous
import functools

import jax
import jax.numpy as jnp
from jax import lax
from jax.experimental import pallas as pl
from jax.experimental.pallas import tpu as pltpu
from jax.experimental.pallas import tpu_sc as plsc

NC, NS, L = 2, 16, 16
NW = NC * NS
UNROLL = 8


@jax.jit
def _sc_embed(xt, tt):
    F, B = xt.shape
    E, V = tt.shape
    e_per_w = E // NW
    n_steps = B // (L * UNROLL)
    mesh = plsc.VectorSubcoreMesh(core_axis_name="c", subcore_axis_name="s")

    @functools.partial(
        pl.kernel,
        out_type=jax.ShapeDtypeStruct((F, E, B), jnp.float32),
        mesh=mesh,
        scratch_types=[
            pltpu.VMEM((V,), jnp.float32),
            pltpu.VMEM((B,), jnp.int32),
            pltpu.VMEM((B,), jnp.int32),
            pltpu.VMEM((B,), jnp.int32),
            pltpu.VMEM((2, 1, B), jnp.float32),
            pltpu.VMEM((2, 1, B), jnp.float32),
            pltpu.SemaphoreType.DMA,
            pltpu.SemaphoreType.DMA,
            pltpu.SemaphoreType.DMA,
        ],
        compiler_params=pltpu.CompilerParams(needs_layout_passes=False),
    )
    def k(xt_hbm, tt_hbm, out_hbm, row_v, idx0, idx1, idx2, ob0, ob1,
          rsem, isem, osem):
        wid = lax.axis_index("s") * NC + lax.axis_index("c")
        idx_bufs = (idx0, idx1, idx2)
        out_bufs = (ob0, ob1)

        def row_copy(j):
            c = pltpu.make_async_copy(
                tt_hbm.at[wid * e_per_w + j], row_v, rsem)
            c.start()
            return c

        def idx_copy(f):
            c = pltpu.make_async_copy(xt_hbm.at[f], idx_bufs[f % 3], isem)
            c.start()
            return c

        def out_copy(p, e):
            c = pltpu.make_async_copy(
                out_bufs[p % 2],
                out_hbm.at[pl.ds(2 * p, 2), pl.ds(e, 1)], osem)
            c.start()
            return c

        P = F // 2
        rc = row_copy(0)
        ics = [idx_copy(0), idx_copy(1)]
        for j in range(e_per_w):
            e = wid * e_per_w + j
            rc.wait()
            if j > 0:
                ics = [idx_copy(0), idx_copy(1)]
            ocs = [None] * P
            for f in range(F):
                p, u = f // 2, f % 2
                ics[f].wait()
                if f + 2 < F:
                    ics.append(idx_copy(f + 2))
                if u == 0 and p >= 2:
                    ocs[p - 2].wait()
                src = idx_bufs[f % 3]
                dst = out_bufs[p % 2].at[u, 0]

                @plsc.parallel_loop(0, B, step=L, unroll=UNROLL)
                def gbody(i, src=src, dst=dst):
                    idx = src[pl.ds(i, L)]
                    dst[pl.ds(i, L)] = plsc.load_gather(row_v, [idx])
                if f == F - 1 and j + 1 < e_per_w:
                    rc = row_copy(j + 1)
                if u == 1:
                    ocs[p] = out_copy(p, e)
            ocs[P - 2].wait()
            ocs[P - 1].wait()

    return k(xt, tt)


def kernel(x, table):
    out_t = _sc_embed(x.T, table.T)
    return out_t.transpose(2, 0, 1)
